# baseline (speedup 1.0000x reference)
import functools

import jax
import jax.numpy as jnp
from jax import lax
from jax.experimental import pallas as pl
from jax.experimental.pallas import tpu as pltpu
from jax.experimental.pallas import tpu_sc as plsc

LANES = 16

_SC_PARAMS = pltpu.CompilerParams(use_tc_tiling_on_sc=False)


def _sc_mesh():
    return plsc.VectorSubcoreMesh(core_axis_name="c", subcore_axis_name="s")


@functools.partial(jax.jit, static_argnums=())
def _sc_build_fused(token_f, pos_f, seg_f):
    VD = token_f.shape[0]
    SD = seg_f.shape[0]
    info = plsc.get_sparse_core_info()
    NW = info.num_cores * info.num_subcores
    D = 64
    S = SD // D
    V = VD // D
    CH = 125
    per_w = V // NW
    nch = per_w // CH
    CHW = CH * D

    @functools.partial(
        pl.kernel,
        mesh=_sc_mesh(),
        compiler_params=_SC_PARAMS,
        out_type=jax.ShapeDtypeStruct((S * VD,), jnp.float32),
        scratch_types=[
            pltpu.VMEM((CHW,), jnp.float32),
            pltpu.VMEM((CHW,), jnp.float32),
            pltpu.VMEM((CHW,), jnp.float32),
            pltpu.VMEM((SD,), jnp.float32),
        ],
    )
    def build(tok_hbm, pos_hbm, seg_hbm, out_hbm, tok_v, pos_v, o_v, seg_v):
        NC = plsc.get_sparse_core_info().num_cores
        wid = lax.axis_index("s") * NC + lax.axis_index("c")
        base_w = wid * (per_w * D)
        pltpu.sync_copy(seg_hbm, seg_v)

        def chunk(ci, carry):
            off = base_w + ci * CHW
            pltpu.sync_copy(tok_hbm.at[pl.ds(off, CHW)], tok_v)
            pltpu.sync_copy(pos_hbm.at[pl.ds(off, CHW)], pos_v)

            def addtp(i, c):
                sl = pl.ds(i * LANES, LANES)
                tok_v[sl] = tok_v[sl] + pos_v[sl]
                return c

            lax.fori_loop(0, CHW // LANES, addtp, 0)
            for s in range(S):
                seg_row = [seg_v[pl.ds(s * D + q * LANES, LANES)]
                           for q in range(D // LANES)]

                def adds(r, c, seg_row=seg_row, s=s):
                    for q in range(D // LANES):
                        sl = pl.ds(r * D + q * LANES, LANES)
                        o_v[sl] = tok_v[sl] + seg_row[q]
                    return c

                lax.fori_loop(0, CH, adds, 0)
                pltpu.sync_copy(o_v, out_hbm.at[pl.ds(s * VD + off, CHW)])
            return carry

        lax.fori_loop(0, nch, chunk, 0)

    return build(token_f, pos_f, seg_f)


@functools.partial(jax.jit, static_argnums=(3, 4))
def _sc_gather(seq_f, lab_f, fused, V, B):
    N = seq_f.shape[0]
    D = fused.shape[1]
    L = N // B
    info = plsc.get_sparse_core_info()
    NC, NS = info.num_cores, info.num_subcores
    NW = NC * NS
    assert B % NW == 0
    b_per_w = B // NW
    BLK = L
    SUB = 128
    nsub = (BLK + SUB - 1) // SUB

    @functools.partial(
        pl.kernel,
        mesh=_sc_mesh(),
        compiler_params=_SC_PARAMS,
        out_type=jax.ShapeDtypeStruct((B, L, D), jnp.float32),
        scratch_types=[
            pltpu.VMEM((BLK,), jnp.int32),
            pltpu.VMEM((BLK,), jnp.int32),
            pltpu.VMEM((BLK,), jnp.int32),
            pltpu.VMEM((BLK, D), jnp.float32),
            pltpu.SemaphoreType.DMA,
        ],
    )
    def gather(seq_hbm, lab_hbm, table_hbm, out_hbm, seq_v, lab_v, idx_v,
               rows_v, sem):
        wid = lax.axis_index("s") * NC + lax.axis_index("c")

        def blk(b, carry):
            batch = wid * b_per_w + b
            base = batch * BLK
            pltpu.sync_copy(seq_hbm.at[pl.ds(base, BLK)], seq_v)
            pltpu.sync_copy(lab_hbm.at[pl.ds(base, BLK)], lab_v)

            def cidx(i, c):
                sl = pl.ds(i * LANES, LANES)
                idx_v[sl] = lab_v[sl] * V + seq_v[sl]
                return c

            lax.fori_loop(0, BLK // LANES, cidx, 0)
            if BLK % LANES:
                sl = pl.ds(BLK - LANES, LANES)
                idx_v[sl] = lab_v[sl] * V + seq_v[sl]
            cps = []
            for j in range(nsub):
                lo = j * SUB
                n = min(SUB, BLK - lo)
                sl = pl.ds(lo, n)
                cps.append(
                    pltpu.async_copy(table_hbm.at[idx_v.at[sl]],
                                     rows_v.at[sl], sem))
            for cp in cps:
                cp.wait()
            pltpu.sync_copy(rows_v, out_hbm.at[batch])
            return carry

        lax.fori_loop(0, b_per_w, blk, 0)

    return gather(seq_f, lab_f, fused)


def kernel(seq, segment_lab, token_table, pos_table, seg_table):
    B, L = seq.shape
    V, D = token_table.shape
    S = seg_table.shape[0]
    fused = _sc_build_fused(token_table.reshape(-1), pos_table.reshape(-1),
                            seg_table.reshape(-1))
    fused = fused.reshape(S * V, D)
    return _sc_gather(seq.reshape(-1), segment_lab.reshape(-1), fused, V, B)


# speedup vs baseline: 1.2607x; 1.2607x over previous
import functools

import jax
import jax.numpy as jnp
from jax import lax
from jax.experimental import pallas as pl
from jax.experimental.pallas import tpu as pltpu
from jax.experimental.pallas import tpu_sc as plsc

LANES = 16


def _fuse_body(token_ref, pos_ref, seg2_ref, out_ref):
    tp = token_ref[...] + pos_ref[...]
    out_ref[...] = tp[None, :, :] + seg2_ref[...][:, None, :]


def _build_fused(token_table, pos_table, seg_table):
    V, D = token_table.shape
    S = seg_table.shape[0]
    W = 2 * D
    tok2 = token_table.reshape(V // 2, W)
    pos2 = pos_table.reshape(V // 2, W)
    seg2 = jnp.concatenate([seg_table, seg_table], axis=-1)
    BV = 1000
    assert (V // 2) % BV == 0
    fused = pl.pallas_call(
        _fuse_body,
        grid=(V // 2 // BV,),
        in_specs=[
            pl.BlockSpec((BV, W), lambda i: (i, 0)),
            pl.BlockSpec((BV, W), lambda i: (i, 0)),
            pl.BlockSpec((S, W), lambda i: (0, 0)),
        ],
        out_specs=pl.BlockSpec((S, BV, W), lambda i: (0, i, 0)),
        out_shape=jax.ShapeDtypeStruct((S, V // 2, W), jnp.float32),
    )(tok2, pos2, seg2)
    return fused.reshape(S * V, D)


@functools.partial(jax.jit, static_argnums=(3,))
def _sc_gather(seq_f, lab_f, fused, V):
    N = seq_f.shape[0]
    D = fused.shape[1]
    info = plsc.get_sparse_core_info()
    NC, NS = info.num_cores, info.num_subcores
    NW = NC * NS
    assert N % NW == 0
    per_w = N // NW
    BLK = 512
    SUB = 128
    assert per_w % BLK == 0 and BLK % SUB == 0
    nblk = per_w // BLK
    mesh = plsc.VectorSubcoreMesh(core_axis_name="c", subcore_axis_name="s")

    @functools.partial(
        pl.kernel,
        mesh=mesh,
        compiler_params=pltpu.CompilerParams(use_tc_tiling_on_sc=False),
        out_type=jax.ShapeDtypeStruct((N, D), jnp.float32),
        scratch_types=[
            pltpu.VMEM((BLK,), jnp.int32),
            pltpu.VMEM((BLK,), jnp.int32),
            pltpu.VMEM((BLK,), jnp.int32),
            pltpu.VMEM((BLK, D), jnp.float32),
            pltpu.SemaphoreType.DMA,
        ],
    )
    def gather(seq_hbm, lab_hbm, table_hbm, out_hbm, seq_v, lab_v, idx_v,
               rows_v, sem):
        wid = lax.axis_index("s") * NC + lax.axis_index("c")

        def blk(b, carry):
            base = wid * per_w + b * BLK
            pltpu.sync_copy(seq_hbm.at[pl.ds(base, BLK)], seq_v)
            pltpu.sync_copy(lab_hbm.at[pl.ds(base, BLK)], lab_v)

            def cidx(i, c):
                sl = pl.ds(i * LANES, LANES)
                idx_v[sl] = lab_v[sl] * V + seq_v[sl]
                return c

            lax.fori_loop(0, BLK // LANES, cidx, 0)
            cps = []
            for j in range(BLK // SUB):
                sl = pl.ds(j * SUB, SUB)
                cps.append(
                    pltpu.async_copy(table_hbm.at[idx_v.at[sl]],
                                     rows_v.at[sl], sem))
            for cp in cps:
                cp.wait()
            pltpu.sync_copy(rows_v, out_hbm.at[pl.ds(base, BLK)])
            return carry

        lax.fori_loop(0, nblk, blk, 0)

    return gather(seq_f, lab_f, fused)


def kernel(seq, segment_lab, token_table, pos_table, seg_table):
    B, L = seq.shape
    V, D = token_table.shape
    fused = _build_fused(token_table, pos_table, seg_table)
    out = _sc_gather(seq.reshape(-1), segment_lab.reshape(-1), fused, V)
    return out.reshape(B, L, D)


# speedup vs baseline: 1.3957x; 1.1072x over previous
import functools

import jax
import jax.numpy as jnp
from jax import lax
from jax.experimental import pallas as pl
from jax.experimental.pallas import tpu as pltpu
from jax.experimental.pallas import tpu_sc as plsc

LANES = 16


def _fuse_body(token_ref, pos_ref, seg2_ref, out_ref):
    tp = token_ref[...] + pos_ref[...]
    out_ref[...] = tp[None, :, :] + seg2_ref[...][:, None, :]


def _build_fused(token_table, pos_table, seg_table):
    V, D = token_table.shape
    S = seg_table.shape[0]
    W = 2 * D
    tok2 = token_table.reshape(V // 2, W)
    pos2 = pos_table.reshape(V // 2, W)
    seg2 = jnp.concatenate([seg_table, seg_table], axis=-1)
    BV = 1000
    assert (V // 2) % BV == 0
    fused = pl.pallas_call(
        _fuse_body,
        grid=(V // 2 // BV,),
        in_specs=[
            pl.BlockSpec((BV, W), lambda i: (i, 0)),
            pl.BlockSpec((BV, W), lambda i: (i, 0)),
            pl.BlockSpec((S, W), lambda i: (0, 0)),
        ],
        out_specs=pl.BlockSpec((S, BV, W), lambda i: (0, i, 0)),
        out_shape=jax.ShapeDtypeStruct((S, V // 2, W), jnp.float32),
    )(tok2, pos2, seg2)
    return fused.reshape(S * V, D)


@functools.partial(jax.jit, static_argnums=(3,))
def _sc_gather(seq_f, lab_f, fused, V):
    N = seq_f.shape[0]
    D = fused.shape[1]
    info = plsc.get_sparse_core_info()
    NC, NS = info.num_cores, info.num_subcores
    NW = NC * NS
    assert N % NW == 0
    per_w = N // NW
    BLK = 512
    SUB = 128
    assert per_w % BLK == 0 and BLK % SUB == 0
    nblk = per_w // BLK
    mesh = plsc.VectorSubcoreMesh(core_axis_name="c", subcore_axis_name="s")

    assert nblk % 2 == 0

    @functools.partial(
        pl.kernel,
        mesh=mesh,
        compiler_params=pltpu.CompilerParams(use_tc_tiling_on_sc=False),
        out_type=jax.ShapeDtypeStruct((N, D), jnp.float32),
        scratch_types=[
            pltpu.VMEM((BLK,), jnp.int32),
            pltpu.VMEM((BLK,), jnp.int32),
            pltpu.VMEM((BLK,), jnp.int32),
            pltpu.VMEM((BLK,), jnp.int32),
            pltpu.VMEM((BLK,), jnp.int32),
            pltpu.VMEM((BLK,), jnp.int32),
            pltpu.VMEM((BLK, D), jnp.float32),
            pltpu.VMEM((BLK, D), jnp.float32),
            pltpu.SemaphoreType.DMA,
            pltpu.SemaphoreType.DMA,
            pltpu.SemaphoreType.DMA,
            pltpu.SemaphoreType.DMA,
        ],
    )
    def gather(seq_hbm, lab_hbm, table_hbm, out_hbm, seq_a, seq_b, lab_a,
               lab_b, idx_a, idx_b, rows_a, rows_b, sio_a, sio_b, sg_a,
               sg_b):
        wid = lax.axis_index("s") * NC + lax.axis_index("c")
        w0 = wid * per_w
        S = (seq_a, seq_b)
        A = (lab_a, lab_b)
        I = (idx_a, idx_b)
        R = (rows_a, rows_b)
        SIO = (sio_a, sio_b)
        SG = (sg_a, sg_b)

        def fire_io(b, p):
            base = w0 + b * BLK
            pltpu.async_copy(seq_hbm.at[pl.ds(base, BLK)], S[p], SIO[p])
            pltpu.async_copy(lab_hbm.at[pl.ds(base, BLK)], A[p], SIO[p])

        def wait_io(b, p):
            base = w0 + b * BLK
            pltpu.make_async_copy(seq_hbm.at[pl.ds(base, BLK)], S[p],
                                  SIO[p]).wait()
            pltpu.make_async_copy(lab_hbm.at[pl.ds(base, BLK)], A[p],
                                  SIO[p]).wait()

        def calc_idx(p):
            def cidx(i, c):
                sl = pl.ds(i * LANES, LANES)
                I[p][sl] = A[p][sl] * V + S[p][sl]
                return c

            lax.fori_loop(0, BLK // LANES, cidx, 0)

        def fire_gathers(p):
            for j in range(BLK // SUB):
                sl = pl.ds(j * SUB, SUB)
                pltpu.async_copy(table_hbm.at[I[p].at[sl]], R[p].at[sl],
                                 SG[p])

        def wait_gathers(p):
            for j in range(BLK // SUB):
                sl = pl.ds(j * SUB, SUB)
                pltpu.make_async_copy(table_hbm.at[I[p].at[sl]],
                                      R[p].at[sl], SG[p]).wait()

        def out_copy(b, p):
            base = w0 + b * BLK
            pltpu.sync_copy(R[p], out_hbm.at[pl.ds(base, BLK)])

        fire_io(0, 0)
        wait_io(0, 0)
        calc_idx(0)
        fire_gathers(0)
        fire_io(1, 1)

        def body(g, carry):
            b1 = 2 * g + 1
            wait_io(b1, 1)
            calc_idx(1)
            fire_gathers(1)
            fire_io(b1 + 1, 0)
            wait_gathers(0)
            out_copy(b1 - 1, 0)
            b2 = b1 + 1
            wait_io(b2, 0)
            calc_idx(0)
            fire_gathers(0)

            @pl.when(b2 + 1 < nblk)
            def _():
                fire_io(b2 + 1, 1)

            wait_gathers(1)
            out_copy(b2 - 1, 1)
            return carry

        lax.fori_loop(0, nblk // 2 - 1, body, 0)
        bl = nblk - 1
        wait_io(bl, 1)
        calc_idx(1)
        fire_gathers(1)
        wait_gathers(0)
        out_copy(bl - 1, 0)
        wait_gathers(1)
        out_copy(bl, 1)

    return gather(seq_f, lab_f, fused)


def kernel(seq, segment_lab, token_table, pos_table, seg_table):
    B, L = seq.shape
    V, D = token_table.shape
    fused = _build_fused(token_table, pos_table, seg_table)
    out = _sc_gather(seq.reshape(-1), segment_lab.reshape(-1), fused, V)
    return out.reshape(B, L, D)


# speedup vs baseline: 1.6673x; 1.1945x over previous
import functools

import jax
import jax.numpy as jnp
from jax import lax
from jax.experimental import pallas as pl
from jax.experimental.pallas import tpu as pltpu
from jax.experimental.pallas import tpu_sc as plsc

LANES = 16


def _fuse_body(token_ref, pos_ref, seg2_ref, out_ref):
    tp = token_ref[...] + pos_ref[...]
    v = tp[None, :, :] + seg2_ref[...][:, None, :]
    out_ref[...] = jnp.concatenate([v, jnp.zeros_like(v)], axis=-1)


def _build_fused(token_table, pos_table, seg_table):
    V, D = token_table.shape
    S = seg_table.shape[0]
    W = 2 * D
    BV = 1000
    assert V % BV == 0
    fused = pl.pallas_call(
        _fuse_body,
        grid=(V // BV,),
        in_specs=[
            pl.BlockSpec((BV, D), lambda i: (i, 0)),
            pl.BlockSpec((BV, D), lambda i: (i, 0)),
            pl.BlockSpec((S, D), lambda i: (0, 0)),
        ],
        out_specs=pl.BlockSpec((S, BV, W), lambda i: (0, i, 0)),
        out_shape=jax.ShapeDtypeStruct((S, V, W), jnp.float32),
    )(token_table, pos_table, seg_table)
    return fused.reshape(S * V, W)


@functools.partial(jax.jit, static_argnums=(3,))
def _sc_gather(seq_f, lab_f, fused, V):
    N = seq_f.shape[0]
    D = fused.shape[1]
    info = plsc.get_sparse_core_info()
    NC, NS = info.num_cores, info.num_subcores
    NW = NC * NS
    assert N % NW == 0
    per_w = N // NW
    BLK = 320
    SUB = 128
    assert per_w % BLK == 0 and BLK % LANES == 0
    nblk = per_w // BLK
    subs = []
    lo = 0
    while lo < BLK:
        subs.append((lo, min(SUB, BLK - lo)))
        lo += SUB
    mesh = plsc.VectorSubcoreMesh(core_axis_name="c", subcore_axis_name="s")

    assert nblk % 2 == 0

    @functools.partial(
        pl.kernel,
        mesh=mesh,
        compiler_params=pltpu.CompilerParams(use_tc_tiling_on_sc=False),
        out_type=jax.ShapeDtypeStruct((N, D), jnp.float32),
        scratch_types=[
            pltpu.VMEM((BLK,), jnp.int32),
            pltpu.VMEM((BLK,), jnp.int32),
            pltpu.VMEM((BLK,), jnp.int32),
            pltpu.VMEM((BLK,), jnp.int32),
            pltpu.VMEM((BLK,), jnp.int32),
            pltpu.VMEM((BLK,), jnp.int32),
            pltpu.VMEM((BLK, D), jnp.float32),
            pltpu.VMEM((BLK, D), jnp.float32),
            pltpu.SemaphoreType.DMA,
            pltpu.SemaphoreType.DMA,
            pltpu.SemaphoreType.DMA,
            pltpu.SemaphoreType.DMA,
        ],
    )
    def gather(seq_hbm, lab_hbm, table_hbm, out_hbm, seq_a, seq_b, lab_a,
               lab_b, idx_a, idx_b, rows_a, rows_b, sio_a, sio_b, sg_a,
               sg_b):
        wid = lax.axis_index("s") * NC + lax.axis_index("c")
        w0 = wid * per_w
        S = (seq_a, seq_b)
        A = (lab_a, lab_b)
        I = (idx_a, idx_b)
        R = (rows_a, rows_b)
        SIO = (sio_a, sio_b)
        SG = (sg_a, sg_b)

        def fire_io(b, p):
            base = w0 + b * BLK
            pltpu.async_copy(seq_hbm.at[pl.ds(base, BLK)], S[p], SIO[p])
            pltpu.async_copy(lab_hbm.at[pl.ds(base, BLK)], A[p], SIO[p])

        def wait_io(b, p):
            base = w0 + b * BLK
            pltpu.make_async_copy(seq_hbm.at[pl.ds(base, BLK)], S[p],
                                  SIO[p]).wait()
            pltpu.make_async_copy(lab_hbm.at[pl.ds(base, BLK)], A[p],
                                  SIO[p]).wait()

        def calc_idx(p):
            def cidx(i, c):
                sl = pl.ds(i * LANES, LANES)
                I[p][sl] = A[p][sl] * V + S[p][sl]
                return c

            lax.fori_loop(0, BLK // LANES, cidx, 0)

        def fire_gathers(p):
            for lo, n in subs:
                sl = pl.ds(lo, n)
                pltpu.async_copy(table_hbm.at[I[p].at[sl]], R[p].at[sl],
                                 SG[p])

        def wait_gathers(p):
            for lo, n in subs:
                sl = pl.ds(lo, n)
                pltpu.make_async_copy(table_hbm.at[I[p].at[sl]],
                                      R[p].at[sl], SG[p]).wait()

        def out_copy(b, p):
            base = w0 + b * BLK
            pltpu.sync_copy(R[p], out_hbm.at[pl.ds(base, BLK)])

        fire_io(0, 0)
        wait_io(0, 0)
        calc_idx(0)
        fire_gathers(0)
        fire_io(1, 1)

        def body(g, carry):
            b1 = 2 * g + 1
            wait_io(b1, 1)
            calc_idx(1)
            fire_gathers(1)
            fire_io(b1 + 1, 0)
            wait_gathers(0)
            out_copy(b1 - 1, 0)
            b2 = b1 + 1
            wait_io(b2, 0)
            calc_idx(0)
            fire_gathers(0)

            @pl.when(b2 + 1 < nblk)
            def _():
                fire_io(b2 + 1, 1)

            wait_gathers(1)
            out_copy(b2 - 1, 1)
            return carry

        lax.fori_loop(0, nblk // 2 - 1, body, 0)
        bl = nblk - 1
        wait_io(bl, 1)
        calc_idx(1)
        fire_gathers(1)
        wait_gathers(0)
        out_copy(bl - 1, 0)
        wait_gathers(1)
        out_copy(bl, 1)

    return gather(seq_f, lab_f, fused)


def kernel(seq, segment_lab, token_table, pos_table, seg_table):
    B, L = seq.shape
    V, D = token_table.shape
    fused = _build_fused(token_table, pos_table, seg_table)
    out = _sc_gather(seq.reshape(-1), segment_lab.reshape(-1), fused, V)
    return out[:, :D].reshape(B, L, D)


# speedup vs baseline: 2.2332x; 1.3394x over previous
import functools

import jax
import jax.numpy as jnp
from jax import lax
from jax.experimental import pallas as pl
from jax.experimental.pallas import tpu as pltpu
from jax.experimental.pallas import tpu_sc as plsc

LANES = 16


def _fuse_body(token_ref, pos_ref, seg2_ref, out_ref):
    tp = token_ref[...] + pos_ref[...]
    out_ref[...] = tp[None, :, :] + seg2_ref[...][:, None, :]


def _build_fused(token_table, pos_table, seg_table):
    V, D = token_table.shape
    S = seg_table.shape[0]
    W = 2 * D
    tok2 = token_table.reshape(V // 2, W)
    pos2 = pos_table.reshape(V // 2, W)
    seg2 = jnp.concatenate([seg_table, seg_table], axis=-1)
    BV = 1000
    assert (V // 2) % BV == 0
    fused = pl.pallas_call(
        _fuse_body,
        grid=(V // 2 // BV,),
        in_specs=[
            pl.BlockSpec((BV, W), lambda i: (i, 0)),
            pl.BlockSpec((BV, W), lambda i: (i, 0)),
            pl.BlockSpec((S, W), lambda i: (0, 0)),
        ],
        out_specs=pl.BlockSpec((S, BV, W), lambda i: (0, i, 0)),
        out_shape=jax.ShapeDtypeStruct((S, V // 2, W), jnp.float32),
    )(tok2, pos2, seg2)
    return fused.reshape(S * V, D)


@functools.partial(jax.jit, static_argnums=(3,))
def _sc_gather(seq_f, lab_f, fused, V):
    N = seq_f.shape[0]
    D = fused.shape[1]
    info = plsc.get_sparse_core_info()
    NC, NS = info.num_cores, info.num_subcores
    NW = NC * NS
    assert N % NW == 0
    per_w = N // NW
    BLK = 512
    SUB = 128
    assert per_w % BLK == 0 and BLK % LANES == 0
    nblk = per_w // BLK
    subs = []
    lo = 0
    while lo < BLK:
        subs.append((lo, min(SUB, BLK - lo)))
        lo += SUB
    mesh = plsc.VectorSubcoreMesh(core_axis_name="c", subcore_axis_name="s")

    assert nblk % 2 == 0

    @functools.partial(
        pl.kernel,
        mesh=mesh,
        compiler_params=pltpu.CompilerParams(use_tc_tiling_on_sc=False),
        out_type=jax.ShapeDtypeStruct((N, 2 * D), jnp.float32),
        scratch_types=[
            pltpu.VMEM((BLK,), jnp.int32),
            pltpu.VMEM((BLK,), jnp.int32),
            pltpu.VMEM((BLK,), jnp.int32),
            pltpu.VMEM((BLK,), jnp.int32),
            pltpu.VMEM((BLK,), jnp.int32),
            pltpu.VMEM((BLK,), jnp.int32),
            pltpu.VMEM((BLK, D), jnp.float32),
            pltpu.VMEM((BLK, D), jnp.float32),
            pltpu.SemaphoreType.DMA,
            pltpu.SemaphoreType.DMA,
            pltpu.SemaphoreType.DMA,
            pltpu.SemaphoreType.DMA,
        ],
    )
    def gather(seq_hbm, lab_hbm, table_hbm, out_hbm, seq_a, seq_b, lab_a,
               lab_b, idx_a, idx_b, rows_a, rows_b, sio_a, sio_b, sg_a,
               sg_b):
        wid = lax.axis_index("s") * NC + lax.axis_index("c")
        w0 = wid * per_w
        S = (seq_a, seq_b)
        A = (lab_a, lab_b)
        I = (idx_a, idx_b)
        R = (rows_a, rows_b)
        SIO = (sio_a, sio_b)
        SG = (sg_a, sg_b)

        def fire_io(b, p):
            base = w0 + b * BLK
            pltpu.async_copy(seq_hbm.at[pl.ds(base, BLK)], S[p], SIO[p])
            pltpu.async_copy(lab_hbm.at[pl.ds(base, BLK)], A[p], SIO[p])

        def wait_io(b, p):
            base = w0 + b * BLK
            pltpu.make_async_copy(seq_hbm.at[pl.ds(base, BLK)], S[p],
                                  SIO[p]).wait()
            pltpu.make_async_copy(lab_hbm.at[pl.ds(base, BLK)], A[p],
                                  SIO[p]).wait()

        def calc_idx(p):
            def cidx(i, c):
                sl = pl.ds(i * LANES, LANES)
                I[p][sl] = A[p][sl] * V + S[p][sl]
                return c

            lax.fori_loop(0, BLK // LANES, cidx, 0)

        def fire_gathers(p):
            for lo, n in subs:
                sl = pl.ds(lo, n)
                pltpu.async_copy(table_hbm.at[I[p].at[sl]], R[p].at[sl],
                                 SG[p])

        def wait_gathers(p):
            for lo, n in subs:
                sl = pl.ds(lo, n)
                pltpu.make_async_copy(table_hbm.at[I[p].at[sl]],
                                      R[p].at[sl], SG[p]).wait()

        def out_copy(b, p):
            base = w0 + b * BLK
            pltpu.sync_copy(R[p], out_hbm.at[pl.ds(base, BLK), pl.ds(0, D)])

        fire_io(0, 0)
        wait_io(0, 0)
        calc_idx(0)
        fire_gathers(0)
        fire_io(1, 1)

        def body(g, carry):
            b1 = 2 * g + 1
            wait_io(b1, 1)
            calc_idx(1)
            fire_gathers(1)
            fire_io(b1 + 1, 0)
            wait_gathers(0)
            out_copy(b1 - 1, 0)
            b2 = b1 + 1
            wait_io(b2, 0)
            calc_idx(0)
            fire_gathers(0)

            @pl.when(b2 + 1 < nblk)
            def _():
                fire_io(b2 + 1, 1)

            wait_gathers(1)
            out_copy(b2 - 1, 1)
            return carry

        lax.fori_loop(0, nblk // 2 - 1, body, 0)
        bl = nblk - 1
        wait_io(bl, 1)
        calc_idx(1)
        fire_gathers(1)
        wait_gathers(0)
        out_copy(bl - 1, 0)
        wait_gathers(1)
        out_copy(bl, 1)

    return gather(seq_f, lab_f, fused)


def kernel(seq, segment_lab, token_table, pos_table, seg_table):
    B, L = seq.shape
    V, D = token_table.shape
    fused = _build_fused(token_table, pos_table, seg_table)
    out = _sc_gather(seq.reshape(-1), segment_lab.reshape(-1), fused, V)
    return out[:, :D].reshape(B, L, D)


# speedup vs baseline: 2.2909x; 1.0258x over previous
import functools

import jax
import jax.numpy as jnp
from jax import lax
from jax.experimental import pallas as pl
from jax.experimental.pallas import tpu as pltpu
from jax.experimental.pallas import tpu_sc as plsc

LANES = 16


def _fuse_body(token_ref, pos_ref, seg2_ref, out_ref):
    tp = token_ref[...] + pos_ref[...]
    out_ref[...] = tp[None, :, :] + seg2_ref[...][:, None, :]


def _build_fused(token_table, pos_table, seg_table):
    V, D = token_table.shape
    S = seg_table.shape[0]
    W = 2 * D
    tok2 = token_table.reshape(V // 2, W)
    pos2 = pos_table.reshape(V // 2, W)
    seg2 = jnp.concatenate([seg_table, seg_table], axis=-1)
    BV = 2000
    assert (V // 2) % BV == 0
    fused = pl.pallas_call(
        _fuse_body,
        grid=(V // 2 // BV,),
        in_specs=[
            pl.BlockSpec((BV, W), lambda i: (i, 0)),
            pl.BlockSpec((BV, W), lambda i: (i, 0)),
            pl.BlockSpec((S, W), lambda i: (0, 0)),
        ],
        out_specs=pl.BlockSpec((S, BV, W), lambda i: (0, i, 0)),
        out_shape=jax.ShapeDtypeStruct((S, V // 2, W), jnp.float32),
    )(tok2, pos2, seg2)
    return fused.reshape(S * V, D)


@functools.partial(jax.jit, static_argnums=(3,))
def _sc_gather(seq_f, lab_f, fused, V):
    N = seq_f.shape[0]
    D = fused.shape[1]
    info = plsc.get_sparse_core_info()
    NC, NS = info.num_cores, info.num_subcores
    NW = NC * NS
    assert N % NW == 0
    per_w = N // NW
    BLK = 640
    SUB = 128
    assert per_w % BLK == 0 and BLK % LANES == 0
    nblk = per_w // BLK
    subs = []
    lo = 0
    while lo < BLK:
        subs.append((lo, min(SUB, BLK - lo)))
        lo += SUB
    mesh = plsc.VectorSubcoreMesh(core_axis_name="c", subcore_axis_name="s")

    assert nblk % 2 == 0

    @functools.partial(
        pl.kernel,
        mesh=mesh,
        compiler_params=pltpu.CompilerParams(use_tc_tiling_on_sc=False),
        out_type=jax.ShapeDtypeStruct((N, 2 * D), jnp.float32),
        scratch_types=[
            pltpu.VMEM((BLK,), jnp.int32),
            pltpu.VMEM((BLK,), jnp.int32),
            pltpu.VMEM((BLK,), jnp.int32),
            pltpu.VMEM((BLK,), jnp.int32),
            pltpu.VMEM((BLK,), jnp.int32),
            pltpu.VMEM((BLK,), jnp.int32),
            pltpu.VMEM((BLK, D), jnp.float32),
            pltpu.VMEM((BLK, D), jnp.float32),
            pltpu.SemaphoreType.DMA,
            pltpu.SemaphoreType.DMA,
            pltpu.SemaphoreType.DMA,
            pltpu.SemaphoreType.DMA,
        ],
    )
    def gather(seq_hbm, lab_hbm, table_hbm, out_hbm, seq_a, seq_b, lab_a,
               lab_b, idx_a, idx_b, rows_a, rows_b, sio_a, sio_b, sg_a,
               sg_b):
        wid = lax.axis_index("s") * NC + lax.axis_index("c")
        w0 = wid * per_w
        S = (seq_a, seq_b)
        A = (lab_a, lab_b)
        I = (idx_a, idx_b)
        R = (rows_a, rows_b)
        SIO = (sio_a, sio_b)
        SG = (sg_a, sg_b)

        def fire_io(b, p):
            base = w0 + b * BLK
            pltpu.async_copy(seq_hbm.at[pl.ds(base, BLK)], S[p], SIO[p])
            pltpu.async_copy(lab_hbm.at[pl.ds(base, BLK)], A[p], SIO[p])

        def wait_io(b, p):
            base = w0 + b * BLK
            pltpu.make_async_copy(seq_hbm.at[pl.ds(base, BLK)], S[p],
                                  SIO[p]).wait()
            pltpu.make_async_copy(lab_hbm.at[pl.ds(base, BLK)], A[p],
                                  SIO[p]).wait()

        def calc_idx(p):
            def cidx(i, c):
                sl = pl.ds(i * LANES, LANES)
                I[p][sl] = A[p][sl] * V + S[p][sl]
                return c

            lax.fori_loop(0, BLK // LANES, cidx, 0)

        def fire_gathers(p):
            for lo, n in subs:
                sl = pl.ds(lo, n)
                pltpu.async_copy(table_hbm.at[I[p].at[sl]], R[p].at[sl],
                                 SG[p])

        def wait_gathers(p):
            for lo, n in subs:
                sl = pl.ds(lo, n)
                pltpu.make_async_copy(table_hbm.at[I[p].at[sl]],
                                      R[p].at[sl], SG[p]).wait()

        def out_copy(b, p):
            base = w0 + b * BLK
            pltpu.sync_copy(R[p], out_hbm.at[pl.ds(base, BLK), pl.ds(0, D)])

        fire_io(0, 0)
        wait_io(0, 0)
        calc_idx(0)
        fire_gathers(0)
        fire_io(1, 1)

        def body(g, carry):
            b1 = 2 * g + 1
            wait_io(b1, 1)
            calc_idx(1)
            fire_gathers(1)
            fire_io(b1 + 1, 0)
            wait_gathers(0)
            out_copy(b1 - 1, 0)
            b2 = b1 + 1
            wait_io(b2, 0)
            calc_idx(0)
            fire_gathers(0)

            @pl.when(b2 + 1 < nblk)
            def _():
                fire_io(b2 + 1, 1)

            wait_gathers(1)
            out_copy(b2 - 1, 1)
            return carry

        lax.fori_loop(0, nblk // 2 - 1, body, 0)
        bl = nblk - 1
        wait_io(bl, 1)
        calc_idx(1)
        fire_gathers(1)
        wait_gathers(0)
        out_copy(bl - 1, 0)
        wait_gathers(1)
        out_copy(bl, 1)

    return gather(seq_f, lab_f, fused)


def kernel(seq, segment_lab, token_table, pos_table, seg_table):
    B, L = seq.shape
    V, D = token_table.shape
    fused = _build_fused(token_table, pos_table, seg_table)
    out = _sc_gather(seq.reshape(-1), segment_lab.reshape(-1), fused, V)
    return out[:, :D].reshape(B, L, D)
